# baseline (speedup 1.0000x reference)
import jax
import jax.numpy as jnp
from jax import lax
from jax.experimental import pallas as pl
from jax.experimental.pallas import tpu as pltpu
from jax.experimental.pallas import tpu_sc as plsc

E = 320000
D = 128
NC = 2
NS = 16
NW = NC * NS
EPW = E // NW
C = 80
NCHUNK = EPW // C
L = 16
LB = 2 * L
W = D
W2 = W // 2


def _body(tab_hbm, src_hbm, dst_hbm, out_hbm,
          idx_s_v, idx_t_v, rs0, rt0, rs1, rt1, out_v, acc_buf,
          sem0, sem1):
    cid = lax.axis_index("c")
    sid = lax.axis_index("s")
    wid = sid * NC + cid

    pltpu.sync_copy(src_hbm.at[wid], idx_s_v)
    pltpu.sync_copy(dst_hbm.at[wid], idx_t_v)

    def issue(j, rs, rt, sem):
        pltpu.async_copy(tab_hbm.at[idx_s_v.at[j]], rs, sem)
        pltpu.async_copy(tab_hbm.at[idx_t_v.at[j]], rt, sem)

    def drain(j, rs, rt, sem):
        pltpu.make_async_copy(tab_hbm.at[idx_s_v.at[j]], rs, sem).wait()
        pltpu.make_async_copy(tab_hbm.at[idx_t_v.at[j]], rt, sem).wait()

    def compute(j, rs, rt):
        def group_body(g, gcarry):
            base = g * L
            for e0 in range(L):
                e = base + e0
                acc_a = None
                acc_b = None
                for k in range(D // LB):
                    vs = plsc.bitcast(rs[e, pl.ds(k * L, L)], jnp.bfloat16)
                    vt = plsc.bitcast(rt[e, pl.ds(W2 + k * L, L)],
                                      jnp.bfloat16)
                    prod = vs * vt
                    a, b = plsc.unpack(
                        prod, format=plsc.PackFormat.INTERLEAVED)
                    acc_a = a if acc_a is None else acc_a + a
                    acc_b = b if acc_b is None else acc_b + b
                acc_buf[pl.ds(e0 * L, L)] = acc_a + acc_b
            lane = lax.iota(jnp.int32, L) * L
            tot = plsc.load_gather(acc_buf, [lane])
            for l in range(1, L):
                tot = tot + plsc.load_gather(acc_buf, [lane + l])
            tot = 1.0 / (1.0 + jnp.exp(-tot))
            out_v[pl.ds(j * C + base, L)] = tot
            return gcarry

        lax.fori_loop(0, C // L, group_body, 0)

    issue(0, rs0, rt0, sem0)

    def pair_body(t, carry):
        j0 = 2 * t
        issue(j0 + 1, rs1, rt1, sem1)
        drain(j0, rs0, rt0, sem0)
        compute(j0, rs0, rt0)
        issue(j0 + 2, rs0, rt0, sem0)
        drain(j0 + 1, rs1, rt1, sem1)
        compute(j0 + 1, rs1, rt1)
        return carry

    lax.fori_loop(0, (NCHUNK - 1) // 2, pair_body, 0)

    drain(NCHUNK - 1, rs0, rt0, sem0)
    compute(NCHUNK - 1, rs0, rt0)

    pltpu.sync_copy(out_v, out_hbm.at[pl.ds(wid * EPW, EPW)])


@jax.jit
def _decode(table, src, dst):
    mesh = plsc.VectorSubcoreMesh(core_axis_name="c", subcore_axis_name="s")
    return pl.kernel(
        _body,
        out_type=jax.ShapeDtypeStruct((E,), jnp.float32),
        mesh=mesh,
        compiler_params=pltpu.CompilerParams(needs_layout_passes=False),
        scratch_types=[
            pltpu.VMEM((NCHUNK, C), jnp.int32),
            pltpu.VMEM((NCHUNK, C), jnp.int32),
            pltpu.VMEM((C, W), jnp.int32),
            pltpu.VMEM((C, W), jnp.int32),
            pltpu.VMEM((C, W), jnp.int32),
            pltpu.VMEM((C, W), jnp.int32),
            pltpu.VMEM((EPW,), jnp.float32),
            pltpu.VMEM((L * L,), jnp.float32),
            pltpu.SemaphoreType.DMA,
            pltpu.SemaphoreType.DMA,
        ],
    )(table, src, dst)


def kernel(x_source, x_target, edge_index):
    ei = edge_index.astype(jnp.int32)
    src = ei[0].reshape(NW, NCHUNK, C)
    dst = ei[1].reshape(NW, NCHUNK, C)
    comb = jnp.concatenate(
        [x_source.astype(jnp.bfloat16), x_target.astype(jnp.bfloat16)],
        axis=1)
    table = jax.lax.bitcast_convert_type(comb.reshape(-1, W, 2), jnp.int32)
    return _decode(table, src, dst)


# speedup vs baseline: 1.2180x; 1.2180x over previous
import jax
import jax.numpy as jnp
from jax import lax
from jax.experimental import pallas as pl
from jax.experimental.pallas import tpu as pltpu
from jax.experimental.pallas import tpu_sc as plsc

E = 320000
D = 128
NC = 2
NS = 16
NW = NC * NS
EPW = E // NW
C = 80
NCHUNK = EPW // C
L = 16


def _body(src_tab_hbm, tgt_tab_hbm, src_hbm, dst_hbm, out_hbm,
          idx_s_v, idx_t_v, rs0, rt0, rs1, rt1, out_v, acc_buf,
          sem0, sem1):
    cid = lax.axis_index("c")
    sid = lax.axis_index("s")
    wid = sid * NC + cid

    pltpu.sync_copy(src_hbm.at[wid], idx_s_v)
    pltpu.sync_copy(dst_hbm.at[wid], idx_t_v)

    def issue(j, rs, rt, sem):
        pltpu.async_copy(src_tab_hbm.at[idx_s_v.at[j]], rs, sem)
        pltpu.async_copy(tgt_tab_hbm.at[idx_t_v.at[j]], rt, sem)

    def drain(j, rs, rt, sem):
        pltpu.make_async_copy(src_tab_hbm.at[idx_s_v.at[j]], rs, sem).wait()
        pltpu.make_async_copy(tgt_tab_hbm.at[idx_t_v.at[j]], rt, sem).wait()

    mask_last = lax.iota(jnp.int32, L) == (L - 1)

    def compute(j, rs, rt):
        def group_body(g, gcarry):
            base = g * L
            for e0 in range(L):
                e = base + e0
                acc = None
                for k in range(D // L):
                    vs = rs[e, pl.ds(k * L, L)]
                    vt = rt[e, pl.ds(k * L, L)]
                    part = vs * vt
                    acc = part if acc is None else acc + part
                plsc.store_compressed(acc_buf.at[pl.ds(e0, L)],
                                      plsc.cumsum(acc), mask=mask_last)
            tot = acc_buf[pl.ds(0, L)]
            tot = 1.0 / (1.0 + jnp.exp(-tot))
            out_v[pl.ds(j * C + base, L)] = tot
            return gcarry

        lax.fori_loop(0, C // L, group_body, 0)

    issue(0, rs0, rt0, sem0)

    def pair_body(t, carry):
        j0 = 2 * t
        issue(j0 + 1, rs1, rt1, sem1)
        drain(j0, rs0, rt0, sem0)
        compute(j0, rs0, rt0)
        issue(j0 + 2, rs0, rt0, sem0)
        drain(j0 + 1, rs1, rt1, sem1)
        compute(j0 + 1, rs1, rt1)
        return carry

    lax.fori_loop(0, (NCHUNK - 1) // 2, pair_body, 0)

    drain(NCHUNK - 1, rs0, rt0, sem0)
    compute(NCHUNK - 1, rs0, rt0)

    pltpu.sync_copy(out_v, out_hbm.at[pl.ds(wid * EPW, EPW)])


@jax.jit
def _decode(src_tab, tgt_tab, src, dst):
    mesh = plsc.VectorSubcoreMesh(core_axis_name="c", subcore_axis_name="s")
    return pl.kernel(
        _body,
        out_type=jax.ShapeDtypeStruct((E,), jnp.float32),
        mesh=mesh,
        compiler_params=pltpu.CompilerParams(needs_layout_passes=False),
        scratch_types=[
            pltpu.VMEM((NCHUNK, C), jnp.int32),
            pltpu.VMEM((NCHUNK, C), jnp.int32),
            pltpu.VMEM((C, D), jnp.float32),
            pltpu.VMEM((C, D), jnp.float32),
            pltpu.VMEM((C, D), jnp.float32),
            pltpu.VMEM((C, D), jnp.float32),
            pltpu.VMEM((EPW,), jnp.float32),
            pltpu.VMEM((2 * L,), jnp.float32),
            pltpu.SemaphoreType.DMA,
            pltpu.SemaphoreType.DMA,
        ],
    )(src_tab, tgt_tab, src, dst)


def kernel(x_source, x_target, edge_index):
    ei = edge_index.astype(jnp.int32)
    src = ei[0].reshape(NW, NCHUNK, C)
    dst = ei[1].reshape(NW, NCHUNK, C)
    return _decode(x_source, x_target, src, dst)
